# baseline (speedup 1.0000x reference)
import jax
import jax.numpy as jnp
from jax import lax
from jax.experimental import pallas as pl
from jax.experimental.pallas import tpu as pltpu
from jax.experimental.pallas import tpu_sc as plsc

B, T, D = 16, 512, 384
F = 4 * T
L = 16
FRAMES = 1024
G = 64
W = G + 8
NCHUNK = FRAMES // G
IPG = G // L
DV = D // L


def _tile_body(enc_hbm, dur_hbm, out_hbm, dur_v, cnt_v, idx_v, wst_v,
               wins, exps, fsems, wsems):
    wid = lax.axis_index("s") * 2 + lax.axis_index("c")
    b = wid // 2
    f0 = (wid % 2) * FRAMES
    i32 = jnp.int32

    pltpu.sync_copy(dur_hbm.at[b], dur_v)

    for m in range(FRAMES // L):
        cnt_v[pl.ds(m * L, L)] = jnp.zeros((L,), i32)

    one_v = jnp.ones((L,), i32)
    zero_v = jnp.zeros((L,), i32)
    run = i32(0)
    base = i32(0)
    for i in range(T // L):
        v = jnp.maximum(dur_v[pl.ds(i * L, L)], 1)
        ends = plsc.cumsum(v) + run
        k = ends - f0
        plsc.addupdate_scatter(cnt_v, [k], one_v,
                               mask=(k >= 0) & (k < FRAMES))
        base = base + jnp.sum(jnp.where(k < 0, one_v, zero_v))
        run = run + jnp.sum(v)

    row0 = base + b * T
    cap = b * T + (T - 1)

    run = row0
    for c in range(NCHUNK):
        for m in range(IPG):
            v = cnt_v[pl.ds((c * IPG + m) * L, L)]
            s = plsc.cumsum(v) + run
            idx_v[c, pl.ds(m * L, L)] = jnp.minimum(s, cap)
            run = run + jnp.sum(v)

    fiota = jax.lax.iota(jnp.int32, L)

    def _expand(c, slot):
        wst = wst_v[c]
        locs = [idx_v[c, pl.ds(g * L, L)] - wst for g in range(IPG)]
        fios = [fiota + g * L for g in range(IPG)]

        def _col(w, carry):
            wsp = zero_v + w
            for g in range(IPG):
                x = plsc.load_gather(wins[slot], [locs[g], wsp])
                plsc.store_scatter(exps[slot], [fios[g], wsp], x)
            return carry
        lax.fori_loop(0, D, _col, i32(0), unroll=8)

    def _fetch(c, slot):
        wst = (idx_v[c, pl.ds(0, L)][0] // 8) * 8
        wst = pl.multiple_of(jnp.minimum(wst, B * T - W), 8)
        wst_v[c] = wst
        pltpu.make_async_copy(enc_hbm.at[pl.ds(wst, W)], wins[slot],
                              fsems[slot]).start()

    def _write(c, slot):
        return pltpu.make_async_copy(
            exps[slot], out_hbm.at[b, pl.ds(f0 + c * G, G)], wsems[slot])

    _fetch(0, 0)
    for c in range(NCHUNK):
        s = c % 2
        if c + 1 < NCHUNK:
            _fetch(c + 1, 1 - s)
        pltpu.make_async_copy(enc_hbm.at[pl.ds(i32(0), W)], wins[s],
                              fsems[s]).wait()
        if c >= 2:
            _write(c - 2, s).wait()
        _expand(c, s)
        _write(c, s).start()
    for c in (NCHUNK - 2, NCHUNK - 1):
        _write(c, c % 2).wait()


@jax.jit
def kernel(encoder_output, durations):
    enc_flat = encoder_output.reshape(B * T, D)
    run = pl.kernel(
        _tile_body,
        out_type=jax.ShapeDtypeStruct((B, F, D), jnp.float32),
        mesh=plsc.VectorSubcoreMesh(core_axis_name="c", subcore_axis_name="s"),
        compiler_params=pltpu.CompilerParams(needs_layout_passes=False),
        scratch_types=[
            pltpu.VMEM((T,), jnp.int32),
            pltpu.VMEM((FRAMES,), jnp.int32),
            pltpu.VMEM((NCHUNK, G + L), jnp.int32),
            pltpu.SMEM((NCHUNK,), jnp.int32),
            [pltpu.VMEM((W, D), jnp.float32) for _ in range(2)],
            [pltpu.VMEM((G, D), jnp.float32) for _ in range(2)],
            [pltpu.SemaphoreType.DMA for _ in range(2)],
            [pltpu.SemaphoreType.DMA for _ in range(2)],
        ],
    )
    return run(enc_flat, durations)


# speedup vs baseline: 3.6489x; 3.6489x over previous
import jax
import jax.numpy as jnp
from jax import lax
from jax.experimental import pallas as pl
from jax.experimental.pallas import tpu as pltpu
from jax.experimental.pallas import tpu_sc as plsc

B, T, D = 16, 512, 384
F = 4 * T
L = 16
FRAMES = 1024
G = 64
NCHUNK = FRAMES // G
IPG = G // L
NBUF = 4


def _tile_body(enc_hbm, dur_hbm, out_hbm, dur_v, cnt_v, idx_v,
               bufs, gsems, wsems):
    wid = lax.axis_index("s") * 2 + lax.axis_index("c")
    b = wid // 2
    f0 = (wid % 2) * FRAMES
    i32 = jnp.int32

    pltpu.sync_copy(dur_hbm.at[b], dur_v)

    for m in range(FRAMES // L):
        cnt_v[pl.ds(m * L, L)] = jnp.zeros((L,), i32)

    one_v = jnp.ones((L,), i32)
    zero_v = jnp.zeros((L,), i32)
    run = i32(0)
    base = i32(0)
    for i in range(T // L):
        v = jnp.maximum(dur_v[pl.ds(i * L, L)], 1)
        ends = plsc.cumsum(v) + run
        k = ends - f0
        plsc.addupdate_scatter(cnt_v, [k], one_v,
                               mask=(k >= 0) & (k < FRAMES))
        base = base + jnp.sum(jnp.where(k < 0, one_v, zero_v))
        run = run + jnp.sum(v)

    row0 = base + b * T
    cap = b * T + (T - 1)

    def _write(c):
        return pltpu.make_async_copy(
            bufs[c % NBUF], out_hbm.at[b, pl.ds(f0 + c * G, G)],
            wsems[c % NBUF])

    run = row0
    for c in range(NCHUNK):
        for m in range(IPG):
            v = cnt_v[pl.ds((c * IPG + m) * L, L)]
            s = plsc.cumsum(v) + run
            idx_v[c, pl.ds(m * L, L)] = jnp.minimum(s, cap)
            run = run + jnp.sum(v)
        if c >= NBUF:
            _write(c - NBUF).wait()
        pltpu.make_async_copy(enc_hbm.at[idx_v.at[c]], bufs[c % NBUF],
                              gsems[c % NBUF]).start()
        if c >= 1:
            pltpu.make_async_copy(enc_hbm.at[idx_v.at[c - 1]],
                                  bufs[(c - 1) % NBUF],
                                  gsems[(c - 1) % NBUF]).wait()
            _write(c - 1).start()
    c = NCHUNK - 1
    pltpu.make_async_copy(enc_hbm.at[idx_v.at[c]], bufs[c % NBUF],
                          gsems[c % NBUF]).wait()
    _write(c).start()
    for c in range(NCHUNK - NBUF, NCHUNK):
        _write(c).wait()


@jax.jit
def kernel(encoder_output, durations):
    enc_flat = encoder_output.reshape(B * T, D)
    run = pl.kernel(
        _tile_body,
        out_type=jax.ShapeDtypeStruct((B, F, D), jnp.float32),
        mesh=plsc.VectorSubcoreMesh(core_axis_name="c", subcore_axis_name="s"),
        compiler_params=pltpu.CompilerParams(needs_layout_passes=False,
                                             use_tc_tiling_on_sc=False),
        scratch_types=[
            pltpu.VMEM((T,), jnp.int32),
            pltpu.VMEM((FRAMES,), jnp.int32),
            pltpu.VMEM((NCHUNK, G), jnp.int32),
            [pltpu.VMEM((G, D), jnp.float32) for _ in range(NBUF)],
            [pltpu.SemaphoreType.DMA for _ in range(NBUF)],
            [pltpu.SemaphoreType.DMA for _ in range(NBUF)],
        ],
    )
    return run(enc_flat, durations)


# speedup vs baseline: 6.8789x; 1.8852x over previous
import jax
import jax.numpy as jnp
from jax import lax
from jax.experimental import pallas as pl
from jax.experimental.pallas import tpu as pltpu
from jax.experimental.pallas import tpu_sc as plsc

B, T, D = 16, 512, 384
F = 4 * T
L = 16
FRAMES = 1024
G = 64
NCHUNK = FRAMES // G
IPG = G // L
NBUF = 4


def _tile_body(enc_hbm, dur_hbm, out_hbm, dur_v, cnt_v, idx_v,
               bufs, gsems, wsems):
    wid = lax.axis_index("s") * 2 + lax.axis_index("c")
    b = wid // 2
    f0 = (wid % 2) * FRAMES
    i32 = jnp.int32

    pltpu.sync_copy(dur_hbm.at[b], dur_v)

    for m in range(FRAMES // L):
        cnt_v[pl.ds(m * L, L)] = jnp.zeros((L,), i32)

    one_v = jnp.ones((L,), i32)
    zero_v = jnp.zeros((L,), i32)
    run = i32(0)
    base = i32(0)
    for i in range(T // L):
        v = jnp.maximum(dur_v[pl.ds(i * L, L)], 1)
        ends = plsc.cumsum(v) + run
        k = ends - f0
        plsc.addupdate_scatter(cnt_v, [k], one_v,
                               mask=(k >= 0) & (k < FRAMES))
        base = base + jnp.sum(jnp.where(k < 0, one_v, zero_v))
        run = run + jnp.sum(v)

    row0 = base + b * T
    cap = b * T + (T - 1)

    def _write(c):
        return pltpu.make_async_copy(
            bufs[c % NBUF], out_hbm.at[b, pl.ds(f0 + c * G, G)],
            wsems[c % NBUF])

    run = row0
    for c in range(NCHUNK):
        for m in range(IPG):
            v = cnt_v[pl.ds((c * IPG + m) * L, L)]
            s = plsc.cumsum(v) + run
            idx_v[c, pl.ds(m * L, L)] = jnp.minimum(s, cap)
            run = run + jnp.sum(v)
        if c >= NBUF:
            _write(c - NBUF).wait()
        pltpu.make_async_copy(enc_hbm.at[idx_v.at[c]], bufs[c % NBUF],
                              gsems[c % NBUF]).start()
        if c >= 1:
            pltpu.make_async_copy(enc_hbm.at[idx_v.at[c - 1]],
                                  bufs[(c - 1) % NBUF],
                                  gsems[(c - 1) % NBUF]).wait()
            _write(c - 1).start()
    c = NCHUNK - 1
    pltpu.make_async_copy(enc_hbm.at[idx_v.at[c]], bufs[c % NBUF],
                          gsems[c % NBUF]).wait()
    _write(c).start()
    for c in range(NCHUNK - NBUF, NCHUNK):
        _write(c).wait()


@jax.jit
def kernel(encoder_output, durations):
    enc_flat = encoder_output.reshape(B * T, D)
    run = pl.kernel(
        _tile_body,
        out_type=jax.ShapeDtypeStruct((B, F, D), jnp.float32),
        mesh=plsc.VectorSubcoreMesh(core_axis_name="c", subcore_axis_name="s"),
        compiler_params=pltpu.CompilerParams(needs_layout_passes=False),
        scratch_types=[
            pltpu.VMEM((T,), jnp.int32),
            pltpu.VMEM((FRAMES,), jnp.int32),
            pltpu.VMEM((NCHUNK, G), jnp.int32),
            [pltpu.VMEM((G, D), jnp.float32) for _ in range(NBUF)],
            [pltpu.SemaphoreType.DMA for _ in range(NBUF)],
            [pltpu.SemaphoreType.DMA for _ in range(NBUF)],
        ],
    )
    return run(enc_flat, durations)


# speedup vs baseline: 7.1575x; 1.0405x over previous
import jax
import jax.numpy as jnp
from jax import lax
from jax.experimental import pallas as pl
from jax.experimental.pallas import tpu as pltpu
from jax.experimental.pallas import tpu_sc as plsc

B, T, D = 16, 512, 384
F = 4 * T
L = 16
FRAMES = 1024
G = 64
NCHUNK = FRAMES // G
IPG = G // L
NBUF = 4


def _tile_body(enc_hbm, dur_hbm, out_hbm, dur_v, cnt_v, idx_v,
               bufs, gsems, wsems):
    wid = lax.axis_index("s") * 2 + lax.axis_index("c")
    b = wid // 2
    f0 = (wid % 2) * FRAMES
    i32 = jnp.int32

    pltpu.sync_copy(dur_hbm.at[b], dur_v)

    for m in range(FRAMES // L):
        cnt_v[pl.ds(m * L, L)] = jnp.zeros((L,), i32)

    one_v = jnp.ones((L,), i32)
    zero_v = jnp.zeros((L,), i32)
    run = i32(0)
    base = i32(0)
    for i in range(T // L):
        v = jnp.maximum(dur_v[pl.ds(i * L, L)], 1)
        ends = plsc.cumsum(v) + run
        k = ends - f0
        plsc.addupdate_scatter(cnt_v, [k], one_v,
                               mask=(k >= 0) & (k < FRAMES))
        base = base + jnp.sum(jnp.where(k < 0, one_v, zero_v))
        run = run + jnp.sum(v)

    row0 = base + b * T
    cap = b * T + (T - 1)

    def _write(c):
        return pltpu.make_async_copy(
            bufs[c % NBUF], out_hbm.at[b, pl.ds(f0 + c * G, G)],
            wsems[c % NBUF])

    run = row0
    for c in range(NCHUNK):
        for m in range(IPG):
            v = cnt_v[pl.ds((c * IPG + m) * L, L)]
            s = plsc.cumsum(v) + run
            idx_v[c, pl.ds(m * L, L)] = jnp.minimum(s, cap)
            run = run + jnp.sum(v)
        if c >= NBUF:
            _write(c - NBUF).wait()
        pltpu.make_async_copy(enc_hbm.at[idx_v.at[c]], bufs[c % NBUF],
                              gsems[c % NBUF]).start()
        if c >= 2:
            pltpu.make_async_copy(enc_hbm.at[idx_v.at[c - 2]],
                                  bufs[(c - 2) % NBUF],
                                  gsems[(c - 2) % NBUF]).wait()
            _write(c - 2).start()
    for c in (NCHUNK - 2, NCHUNK - 1):
        pltpu.make_async_copy(enc_hbm.at[idx_v.at[c]], bufs[c % NBUF],
                              gsems[c % NBUF]).wait()
        _write(c).start()
    for c in range(NCHUNK - NBUF, NCHUNK):
        _write(c).wait()


@jax.jit
def kernel(encoder_output, durations):
    enc_flat = encoder_output.reshape(B * T, D)
    run = pl.kernel(
        _tile_body,
        out_type=jax.ShapeDtypeStruct((B, F, D), jnp.float32),
        mesh=plsc.VectorSubcoreMesh(core_axis_name="c", subcore_axis_name="s"),
        compiler_params=pltpu.CompilerParams(needs_layout_passes=False),
        scratch_types=[
            pltpu.VMEM((T,), jnp.int32),
            pltpu.VMEM((FRAMES,), jnp.int32),
            pltpu.VMEM((NCHUNK, G), jnp.int32),
            [pltpu.VMEM((G, D), jnp.float32) for _ in range(NBUF)],
            [pltpu.SemaphoreType.DMA for _ in range(NBUF)],
            [pltpu.SemaphoreType.DMA for _ in range(NBUF)],
        ],
    )
    return run(enc_flat, durations)


# speedup vs baseline: 7.3267x; 1.0236x over previous
import jax
import jax.numpy as jnp
from jax import lax
from jax.experimental import pallas as pl
from jax.experimental.pallas import tpu as pltpu
from jax.experimental.pallas import tpu_sc as plsc

B, T, D = 16, 512, 384
F = 4 * T
L = 16
FRAMES = 1024
G = 64
NCHUNK = FRAMES // G
IPG = G // L
NBUF = 5


def _tile_body(enc_hbm, dur_hbm, out_hbm, dur_v, cnt_v, idx_v,
               bufs, gsems, wsems):
    wid = lax.axis_index("s") * 2 + lax.axis_index("c")
    b = wid // 2
    f0 = (wid % 2) * FRAMES
    i32 = jnp.int32

    pltpu.sync_copy(dur_hbm.at[b], dur_v)

    for m in range(FRAMES // L):
        cnt_v[pl.ds(m * L, L)] = jnp.zeros((L,), i32)

    one_v = jnp.ones((L,), i32)
    zero_v = jnp.zeros((L,), i32)
    run = i32(0)
    base = i32(0)
    for i in range(T // L):
        v = jnp.maximum(dur_v[pl.ds(i * L, L)], 1)
        ends = plsc.cumsum(v) + run
        k = ends - f0
        plsc.addupdate_scatter(cnt_v, [k], one_v,
                               mask=(k >= 0) & (k < FRAMES))
        base = base + jnp.sum(jnp.where(k < 0, one_v, zero_v))
        run = run + jnp.sum(v)

    row0 = base + b * T
    cap = b * T + (T - 1)

    def _write(c):
        return pltpu.make_async_copy(
            bufs[c % NBUF], out_hbm.at[b, pl.ds(f0 + c * G, G)],
            wsems[c % NBUF])

    run = row0
    for c in range(NCHUNK):
        for m in range(IPG):
            v = cnt_v[pl.ds((c * IPG + m) * L, L)]
            s = plsc.cumsum(v) + run
            idx_v[c, pl.ds(m * L, L)] = jnp.minimum(s, cap)
            run = run + jnp.sum(v)
        if c >= NBUF:
            _write(c - NBUF).wait()
        pltpu.make_async_copy(enc_hbm.at[idx_v.at[c]], bufs[c % NBUF],
                              gsems[c % NBUF]).start()
        if c >= 3:
            pltpu.make_async_copy(enc_hbm.at[idx_v.at[c - 3]],
                                  bufs[(c - 3) % NBUF],
                                  gsems[(c - 3) % NBUF]).wait()
            _write(c - 3).start()
    for c in (NCHUNK - 3, NCHUNK - 2, NCHUNK - 1):
        pltpu.make_async_copy(enc_hbm.at[idx_v.at[c]], bufs[c % NBUF],
                              gsems[c % NBUF]).wait()
        _write(c).start()
    for c in range(NCHUNK - NBUF, NCHUNK):
        _write(c).wait()


@jax.jit
def kernel(encoder_output, durations):
    enc_flat = encoder_output.reshape(B * T, D)
    run = pl.kernel(
        _tile_body,
        out_type=jax.ShapeDtypeStruct((B, F, D), jnp.float32),
        mesh=plsc.VectorSubcoreMesh(core_axis_name="c", subcore_axis_name="s"),
        compiler_params=pltpu.CompilerParams(needs_layout_passes=False),
        scratch_types=[
            pltpu.VMEM((T,), jnp.int32),
            pltpu.VMEM((FRAMES,), jnp.int32),
            pltpu.VMEM((NCHUNK, G), jnp.int32),
            [pltpu.VMEM((G, D), jnp.float32) for _ in range(NBUF)],
            [pltpu.SemaphoreType.DMA for _ in range(NBUF)],
            [pltpu.SemaphoreType.DMA for _ in range(NBUF)],
        ],
    )
    return run(enc_flat, durations)


# speedup vs baseline: 7.4061x; 1.0108x over previous
import jax
import jax.numpy as jnp
from jax import lax
from jax.experimental import pallas as pl
from jax.experimental.pallas import tpu as pltpu
from jax.experimental.pallas import tpu_sc as plsc

B, T, D = 16, 512, 384
F = 4 * T
L = 16
FRAMES = 1024
G = 64
NCHUNK = FRAMES // G
IPG = G // L
NBUF = 5


def _tile_body(enc_hbm, dur_hbm, out_hbm, dur_v, cnt_v, idx_v,
               bufs, gsems, wsems):
    wid = lax.axis_index("s") * 2 + lax.axis_index("c")
    b = wid // 2
    f0 = (wid % 2) * FRAMES
    i32 = jnp.int32

    pltpu.sync_copy(dur_hbm.at[b], dur_v)

    for m in range(FRAMES // L):
        cnt_v[pl.ds(m * L, L)] = jnp.zeros((L,), i32)

    one_v = jnp.ones((L,), i32)
    zero_v = jnp.zeros((L,), i32)
    run = i32(0)
    base = i32(0)
    for i in range(T // L):
        v = jnp.maximum(dur_v[pl.ds(i * L, L)], 1)
        ends = plsc.cumsum(v) + run
        k = ends - f0
        plsc.addupdate_scatter(cnt_v, [k], one_v,
                               mask=(k >= 0) & (k < FRAMES))
        base = base + jnp.sum(jnp.where(k < 0, one_v, zero_v))
        run = run + jnp.sum(v)

    row0 = base + b * T
    cap = b * T + (T - 1)

    def _write(c):
        return pltpu.make_async_copy(
            bufs[c % NBUF], out_hbm.at[b, pl.ds(f0 + c * G, G)],
            wsems[c % NBUF])

    run = row0
    for c in range(NCHUNK):
        for m in range(IPG):
            v = cnt_v[pl.ds((c * IPG + m) * L, L)]
            s = plsc.cumsum(v) + run
            idx_v[c, pl.ds(m * L, L)] = jnp.minimum(s, cap)
            run = run + jnp.sum(v)
        if c >= NBUF:
            _write(c - NBUF).wait()
        pltpu.make_async_copy(enc_hbm.at[idx_v.at[c]], bufs[c % NBUF],
                              gsems[c % NBUF]).start()
        if c >= 4:
            pltpu.make_async_copy(enc_hbm.at[idx_v.at[c - 4]],
                                  bufs[(c - 4) % NBUF],
                                  gsems[(c - 4) % NBUF]).wait()
            _write(c - 4).start()
    for c in (NCHUNK - 4, NCHUNK - 3, NCHUNK - 2, NCHUNK - 1):
        pltpu.make_async_copy(enc_hbm.at[idx_v.at[c]], bufs[c % NBUF],
                              gsems[c % NBUF]).wait()
        _write(c).start()
    for c in range(NCHUNK - NBUF, NCHUNK):
        _write(c).wait()


@jax.jit
def kernel(encoder_output, durations):
    enc_flat = encoder_output.reshape(B * T, D)
    run = pl.kernel(
        _tile_body,
        out_type=jax.ShapeDtypeStruct((B, F, D), jnp.float32),
        mesh=plsc.VectorSubcoreMesh(core_axis_name="c", subcore_axis_name="s"),
        compiler_params=pltpu.CompilerParams(needs_layout_passes=False),
        scratch_types=[
            pltpu.VMEM((T,), jnp.int32),
            pltpu.VMEM((FRAMES,), jnp.int32),
            pltpu.VMEM((NCHUNK, G), jnp.int32),
            [pltpu.VMEM((G, D), jnp.float32) for _ in range(NBUF)],
            [pltpu.SemaphoreType.DMA for _ in range(NBUF)],
            [pltpu.SemaphoreType.DMA for _ in range(NBUF)],
        ],
    )
    return run(enc_flat, durations)
